# baseline (speedup 1.0000x reference)
import functools

import jax
import jax.numpy as jnp
from jax import lax
from jax.experimental import pallas as pl
from jax.experimental.pallas import tpu as pltpu
from jax.experimental.pallas import tpu_sc as plsc

B = 4096
L = 200
VOCAB = 100000
EMBED = 64
LANES = 16
CHUNK = 8
NCHUNK = L // CHUNK
NBUF = 4

NW = 32
SEQ_PER_W = B // NW


@functools.partial(
    pl.kernel,
    out_type=jax.ShapeDtypeStruct((B, EMBED), jnp.float32),
    mesh=plsc.VectorSubcoreMesh(core_axis_name="c", subcore_axis_name="s"),
    compiler_params=pltpu.CompilerParams(
        use_tc_tiling_on_sc=False, needs_layout_passes=False
    ),
    scratch_types=[
        pltpu.VMEM((SEQ_PER_W, L), jnp.int32),
        pltpu.VMEM((NBUF, L, EMBED // 2), jnp.uint32),
        pltpu.VMEM((SEQ_PER_W, EMBED), jnp.float32),
        pltpu.SemaphoreType.DMA,
        pltpu.SemaphoreType.DMA,
        pltpu.SemaphoreType.DMA,
        pltpu.SemaphoreType.DMA,
    ],
)
def _embed_mean(ids_hbm, table_hbm, out_hbm, idx_v, rbuf, out_v,
                sem0, sem1, sem2, sem3):
    wid = lax.axis_index("s") * 2 + lax.axis_index("c")
    base = wid * SEQ_PER_W
    bufs = tuple(rbuf.at[b] for b in range(NBUF))
    sems = (sem0, sem1, sem2, sem3)

    pltpu.sync_copy(ids_hbm.at[pl.ds(base, SEQ_PER_W)], idx_v)

    def fire(b, s):
        row = jnp.minimum(s, SEQ_PER_W - 1)
        pltpu.async_copy(table_hbm.at[idx_v.at[row]], bufs[b], sems[b])

    def wait(b):
        pltpu.make_async_copy(table_hbm.at[pl.ds(0, L)], bufs[b], sems[b]).wait()

    for b in range(NBUF):
        fire(b, b)

    scale = jnp.float32(1.0 / L)
    cols = (0, 32, 16, 48)

    def seq_step(b, s):
        wait(b)
        buf = bufs[b]

        def red(i, acc):
            l0 = i * CHUNK
            half = []
            for c in range(2):
                r = [
                    plsc.bitcast(buf[l0 + j, pl.ds(16 * c, 16)], jnp.bfloat16)
                    for j in range(CHUNK)
                ]
                while len(r) > 1:
                    r = [r[k] + r[k + 1] for k in range(0, len(r), 2)]
                half.append(r[0])
            u0a, u0b = plsc.unpack(half[0], format=plsc.PackFormat.INTERLEAVED)
            u1a, u1b = plsc.unpack(half[1], format=plsc.PackFormat.INTERLEAVED)
            return (acc[0] + u0a, acc[1] + u0b, acc[2] + u1a, acc[3] + u1b)

        acc = lax.fori_loop(
            0, NCHUNK, red,
            tuple(jnp.zeros((LANES,), jnp.float32) for _ in range(4)),
        )
        for e in range(4):
            out_v[s, pl.ds(cols[e], LANES)] = acc[e] * scale
        fire(b, s + NBUF)

    def grp_body(g, carry):
        for b in range(NBUF):
            seq_step(b, NBUF * g + b)
        return carry

    lax.fori_loop(0, SEQ_PER_W // NBUF, grp_body, 0)
    for b in range(NBUF):
        wait(b)
    pltpu.sync_copy(out_v, out_hbm.at[pl.ds(base, SEQ_PER_W)])


VBLK = 256
OB = VBLK // 4


def _pack_body(tt_ref, out_ref):
    x = tt_ref[...].astype(jnp.bfloat16)
    row = jax.lax.broadcasted_iota(jnp.int32, (OB, VBLK), 0)
    col = jax.lax.broadcasted_iota(jnp.int32, (OB, VBLK), 1)
    for a in range(4):
        pi = jnp.where(col == 4 * row + a, 1.0, 0.0).astype(jnp.bfloat16)
        y = jax.lax.dot_general(
            pi, x, (((1,), (1,)), ((), ())),
            preferred_element_type=jnp.float32,
        )
        bits = jax.lax.bitcast_convert_type(y, jnp.uint32)
        packed = (bits[:, : EMBED // 2] >> 16) | (
            bits[:, EMBED // 2 :] & jnp.uint32(0xFFFF0000)
        )
        out_ref[:, 32 * a : 32 * (a + 1)] = packed


_pack_table = pl.pallas_call(
    _pack_body,
    grid=(pl.cdiv(VOCAB, VBLK),),
    in_specs=[pl.BlockSpec((EMBED, VBLK), lambda i: (0, i))],
    out_specs=pl.BlockSpec((OB, 128), lambda i: (i, 0)),
    out_shape=jax.ShapeDtypeStruct((VOCAB // 4, 128), jnp.uint32),
)


def kernel(kmer_ids, table):
    packed = _pack_table(table.T).reshape(VOCAB, EMBED // 2)
    return _embed_mean(kmer_ids, packed)


# speedup vs baseline: 1.9670x; 1.9670x over previous
import functools

import jax
import jax.numpy as jnp
from jax import lax
from jax.experimental import pallas as pl
from jax.experimental.pallas import tpu as pltpu
from jax.experimental.pallas import tpu_sc as plsc

B = 4096
L = 200
VOCAB = 100000
EMBED = 64
LANES = 16
CHUNK = 8
NCHUNK = L // CHUNK
NBUF = 4

NW = 32
SEQ_PER_W = B // NW


@functools.partial(
    pl.kernel,
    out_type=jax.ShapeDtypeStruct((B, EMBED), jnp.float32),
    mesh=plsc.VectorSubcoreMesh(core_axis_name="c", subcore_axis_name="s"),
    compiler_params=pltpu.CompilerParams(
        use_tc_tiling_on_sc=False, needs_layout_passes=False
    ),
    scratch_types=[
        pltpu.VMEM((SEQ_PER_W, L), jnp.int32),
        pltpu.VMEM((NBUF, L, EMBED), jnp.bfloat16),
        pltpu.VMEM((SEQ_PER_W, EMBED), jnp.float32),
        pltpu.SemaphoreType.DMA,
        pltpu.SemaphoreType.DMA,
        pltpu.SemaphoreType.DMA,
        pltpu.SemaphoreType.DMA,
    ],
)
def _embed_mean(ids_hbm, table_hbm, out_hbm, idx_v, rbuf, out_v,
                sem0, sem1, sem2, sem3):
    wid = lax.axis_index("s") * 2 + lax.axis_index("c")
    base = wid * SEQ_PER_W
    bufs = tuple(rbuf.at[b] for b in range(NBUF))
    sems = (sem0, sem1, sem2, sem3)

    pltpu.sync_copy(ids_hbm.at[pl.ds(base, SEQ_PER_W)], idx_v)

    def fire(b, s):
        row = jnp.minimum(s, SEQ_PER_W - 1)
        pltpu.async_copy(table_hbm.at[idx_v.at[row]], bufs[b], sems[b])

    def wait(b):
        pltpu.make_async_copy(table_hbm.at[pl.ds(0, L)], bufs[b], sems[b]).wait()

    for b in range(NBUF):
        fire(b, b)

    scale = jnp.float32(1.0 / L)
    iota = lax.iota(jnp.int32, LANES)
    cols = (2 * iota, 2 * iota + 1, 2 * iota + 32, 2 * iota + 33)

    def seq_step(b, s):
        wait(b)
        buf = bufs[b]

        def red(i, acc):
            l0 = i * CHUNK
            half = []
            for c in range(2):
                r = [buf[l0 + j, pl.ds(32 * c, 32)] for j in range(CHUNK)]
                while len(r) > 1:
                    r = [r[k] + r[k + 1] for k in range(0, len(r), 2)]
                half.append(r[0])
            u0a, u0b = plsc.unpack(half[0], format=plsc.PackFormat.INTERLEAVED)
            u1a, u1b = plsc.unpack(half[1], format=plsc.PackFormat.INTERLEAVED)
            return (acc[0] + u0a, acc[1] + u0b, acc[2] + u1a, acc[3] + u1b)

        acc = lax.fori_loop(
            0, NCHUNK, red,
            tuple(jnp.zeros((LANES,), jnp.float32) for _ in range(4)),
        )
        row_idx = iota * 0 + s
        for e in range(4):
            plsc.store_scatter(out_v, [row_idx, cols[e]], acc[e] * scale)
        fire(b, s + NBUF)

    def grp_body(g, carry):
        for b in range(NBUF):
            seq_step(b, NBUF * g + b)
        return carry

    lax.fori_loop(0, SEQ_PER_W // NBUF, grp_body, 0)
    for b in range(NBUF):
        wait(b)
    pltpu.sync_copy(out_v, out_hbm.at[pl.ds(base, SEQ_PER_W)])


def kernel(kmer_ids, table):
    table_bf = table.astype(jnp.bfloat16)
    return _embed_mean(kmer_ids, table_bf)
